# baseline (speedup 1.0000x reference)
import functools

import jax
import jax.numpy as jnp
from jax import lax
from jax.experimental import pallas as pl
from jax.experimental.pallas import tpu as pltpu
from jax.experimental.pallas import tpu_sc as plsc

_L = 16
_B, _T, _D = 16, 2048, 512
_NC, _NS = 2, 16
_NW = _NC * _NS
_HALF = _T // 2
_CHUNK = 64
_NCHUNK = _HALF // _CHUNK
_VPC = _CHUNK // _L
_NVEC = _T // _L
_WVEC = _HALF // _L


def _reps(d):
    d = jnp.minimum(jnp.maximum(d, 0.0), 4096.0)
    return (d + 0.5).astype(jnp.int32)


def _sc_body(x_hbm, dur_hbm, out_hbm, meta_hbm,
             dur_v, ind_v, g_v, meta_v, buf0, buf1, rs0, rs1, ws0, ws1):
    cid = lax.axis_index("c")
    sid = lax.axis_index("s")
    wid = sid * _NC + cid
    row = wid // 2
    half = wid % 2

    pltpu.sync_copy(dur_hbm.at[row], dur_v)

    row_base = row * _T
    lo = half * _WVEC
    out_base = row_base + half * _HALF

    def allone_body(k, acc):
        r = _reps(dur_v[pl.ds(k * _L, _L)])
        return jnp.logical_and(acc, jnp.all(r == 1))

    allone = lax.fori_loop(0, _NVEC, allone_body, True)

    def fast_path(_):
        return 1, out_base

    def general_path(_):
        zeros = jnp.zeros((_L,), jnp.int32)

        def zero_body(k, carry):
            ind_v[pl.ds(k * _L, _L)] = zeros
            return carry

        lax.fori_loop(0, _NVEC, zero_body, 0)

        ones = jnp.ones((_L,), jnp.int32)

        def scat_body(k, carry):
            r = _reps(dur_v[pl.ds(k * _L, _L)])
            incl = jnp.cumsum(r)
            excl = incl - r + carry
            plsc.addupdate_scatter(ind_v, (excl,), ones, mask=excl < _T)
            return carry + jnp.sum(r)

        lax.fori_loop(0, _NVEC, scat_body, 0)

        def presum_body(k, carry):
            return carry + jnp.sum(ind_v[pl.ds(k * _L, _L)])

        carry = lax.fori_loop(0, lo, presum_body, 0)

        def contig_body(k, acc):
            ind = ind_v[pl.ds(k * _L, _L)]
            return jnp.logical_and(acc, jnp.all(ind == 1))

        contig = lax.fori_loop(lo, lo + _WVEC, contig_body, True)

        def contig_path(_):
            return 1, row_base + carry

        contig = jnp.logical_and(contig, carry % 8 == 0)

        def ragged_path(_):
            bufs, rsems, wsems = (buf0, buf1), (rs0, rs1), (ws0, ws1)
            gathers = [None, None]
            writes = [None, None]
            ccarry = carry

            def out_slice(cc):
                return out_hbm.at[pl.ds(out_base + cc * _CHUNK, _CHUNK)]

            for cc in range(_NCHUNK):
                b = cc % 2
                if writes[b] is not None:
                    writes[b].wait()
                for o in range(_VPC):
                    k = lo + cc * _VPC + o
                    ind = ind_v[pl.ds(k * _L, _L)]
                    cs = jnp.cumsum(ind) + ccarry
                    g_v[cc, pl.ds(o * _L, _L)] = cs - 1 + row_base
                    ccarry = ccarry + jnp.sum(ind)
                gathers[b] = pltpu.async_copy(
                    x_hbm.at[g_v.at[cc]], bufs[b], rsems[b])
                if cc >= 1:
                    pb = 1 - b
                    gathers[pb].wait()
                    writes[pb] = pltpu.async_copy(
                        bufs[pb], out_slice(cc - 1), wsems[pb])

            last = (_NCHUNK - 1) % 2
            gathers[last].wait()
            writes[last] = pltpu.async_copy(
                bufs[last], out_slice(_NCHUNK - 1), wsems[last])
            writes[0].wait()
            writes[1].wait()
            return 0, 0

        return lax.cond(contig, contig_path, ragged_path, 0)

    ok, start = lax.cond(allone, fast_path, general_path, 0)

    lane = lax.iota(jnp.int32, _L)
    meta_v[...] = jnp.where(lane == 0, ok, jnp.where(lane == 1, start, 0))
    pltpu.sync_copy(meta_v, meta_hbm.at[wid])


_sc_stage = functools.partial(
    pl.kernel,
    out_type=(
        jax.ShapeDtypeStruct((_B * _T, _D), jnp.float32),
        jax.ShapeDtypeStruct((_NW, _L), jnp.int32),
    ),
    mesh=plsc.VectorSubcoreMesh(
        core_axis_name="c", subcore_axis_name="s",
        num_cores=_NC, num_subcores=_NS),
    compiler_params=pltpu.CompilerParams(needs_layout_passes=False),
    scratch_types=[
        pltpu.VMEM((_T,), jnp.float32),
        pltpu.VMEM((_T,), jnp.int32),
        pltpu.VMEM((_NCHUNK, _CHUNK), jnp.int32),
        pltpu.VMEM((_L,), jnp.int32),
        pltpu.VMEM((_CHUNK, _D), jnp.float32),
        pltpu.VMEM((_CHUNK, _D), jnp.float32),
        pltpu.SemaphoreType.DMA,
        pltpu.SemaphoreType.DMA,
        pltpu.SemaphoreType.DMA,
        pltpu.SemaphoreType.DMA,
    ],
)(_sc_body)


def _tc_body(meta_ref, x_ref, part_ref, out_ref, b0, b1, s0, s1, t0, t1):
    del part_ref
    bufs, sin, sout = (b0, b1), (s0, s1), (t0, t1)
    oks, descs_in, descs_out = [], [], []
    for c in range(_NW):
        b = c % 2
        oks.append(meta_ref[c, 0] == 1)
        descs_in.append(pltpu.make_async_copy(
            x_ref.at[pl.ds(pl.multiple_of(meta_ref[c, 1], 8), _HALF)],
            bufs[b], sin[b]))
        descs_out.append(pltpu.make_async_copy(
            bufs[b], out_ref.at[pl.ds(c * _HALF, _HALF)], sout[b]))

    def when_do(pred, fn):
        pl.when(pred)(fn)

    for c in range(_NW):
        if c >= 2:
            when_do(oks[c - 2], lambda c=c: descs_out[c - 2].wait())
        when_do(oks[c], lambda c=c: descs_in[c].start())
        if c >= 1:
            def retire(c=c):
                descs_in[c - 1].wait()
                descs_out[c - 1].start()
            when_do(oks[c - 1], retire)

    def retire_last():
        descs_in[_NW - 1].wait()
        descs_out[_NW - 1].start()

    when_do(oks[_NW - 1], retire_last)
    when_do(oks[_NW - 2], lambda: descs_out[_NW - 2].wait())
    when_do(oks[_NW - 1], lambda: descs_out[_NW - 1].wait())


_tc_stage = pl.pallas_call(
    _tc_body,
    out_shape=jax.ShapeDtypeStruct((_B * _T, _D), jnp.float32),
    in_specs=[
        pl.BlockSpec(memory_space=pltpu.SMEM),
        pl.BlockSpec(memory_space=pl.ANY),
        pl.BlockSpec(memory_space=pl.ANY),
    ],
    out_specs=pl.BlockSpec(memory_space=pl.ANY),
    scratch_shapes=[
        pltpu.VMEM((_HALF, _D), jnp.float32),
        pltpu.VMEM((_HALF, _D), jnp.float32),
        pltpu.SemaphoreType.DMA,
        pltpu.SemaphoreType.DMA,
        pltpu.SemaphoreType.DMA,
        pltpu.SemaphoreType.DMA,
    ],
    input_output_aliases={2: 0},
)


def kernel(x, dur):
    xr = x.reshape(_B * _T, _D)
    partial, meta = _sc_stage(xr, dur)
    out = _tc_stage(meta, xr, partial)
    return out.reshape(_B, _T, _D)


# speedup vs baseline: 1.0076x; 1.0076x over previous
import functools

import jax
import jax.numpy as jnp
from jax import lax
from jax.experimental import pallas as pl
from jax.experimental.pallas import tpu as pltpu
from jax.experimental.pallas import tpu_sc as plsc

_L = 16
_B, _T, _D = 16, 2048, 512
_NC, _NS = 2, 16
_NW = _NC * _NS
_HALF = _T // 2
_CHUNK = 64
_NCHUNK = _HALF // _CHUNK
_VPC = _CHUNK // _L
_NVEC = _T // _L
_WVEC = _HALF // _L


def _reps(d):
    d = jnp.minimum(jnp.maximum(d, 0.0), 4096.0)
    return (d + 0.5).astype(jnp.int32)


def _sc_body(x_hbm, dur_hbm, out_hbm, meta_hbm,
             dur_v, ind_v, g_v, meta_v, buf0, buf1, rs0, rs1, ws0, ws1):
    cid = lax.axis_index("c")
    sid = lax.axis_index("s")
    wid = sid * _NC + cid
    row = wid // 2
    half = wid % 2

    pltpu.sync_copy(dur_hbm.at[row], dur_v)

    row_base = row * _T
    lo = half * _WVEC
    out_base = row_base + half * _HALF

    def allone_body(k, acc):
        r = _reps(dur_v[pl.ds(k * _L, _L)])
        return jnp.logical_and(acc, jnp.all(r == 1))

    allone = lax.fori_loop(0, _NVEC, allone_body, True)

    def fast_path(_):
        return 1, out_base

    def general_path(_):
        zeros = jnp.zeros((_L,), jnp.int32)

        def zero_body(k, carry):
            ind_v[pl.ds(k * _L, _L)] = zeros
            return carry

        lax.fori_loop(0, _NVEC, zero_body, 0)

        ones = jnp.ones((_L,), jnp.int32)

        def scat_body(k, carry):
            r = _reps(dur_v[pl.ds(k * _L, _L)])
            incl = jnp.cumsum(r)
            excl = incl - r + carry
            plsc.addupdate_scatter(ind_v, (excl,), ones, mask=excl < _T)
            return carry + jnp.sum(r)

        lax.fori_loop(0, _NVEC, scat_body, 0)

        def presum_body(k, carry):
            return carry + jnp.sum(ind_v[pl.ds(k * _L, _L)])

        carry = lax.fori_loop(0, lo, presum_body, 0)

        def contig_body(k, acc):
            ind = ind_v[pl.ds(k * _L, _L)]
            return jnp.logical_and(acc, jnp.all(ind == 1))

        contig = lax.fori_loop(lo, lo + _WVEC, contig_body, True)

        def contig_path(_):
            return 1, row_base + carry

        contig = jnp.logical_and(contig, carry % 8 == 0)

        def ragged_path(_):
            bufs, rsems, wsems = (buf0, buf1), (rs0, rs1), (ws0, ws1)
            gathers = [None, None]
            writes = [None, None]
            ccarry = carry

            def out_slice(cc):
                return out_hbm.at[pl.ds(out_base + cc * _CHUNK, _CHUNK)]

            for cc in range(_NCHUNK):
                b = cc % 2
                if writes[b] is not None:
                    writes[b].wait()
                for o in range(_VPC):
                    k = lo + cc * _VPC + o
                    ind = ind_v[pl.ds(k * _L, _L)]
                    cs = jnp.cumsum(ind) + ccarry
                    g_v[cc, pl.ds(o * _L, _L)] = cs - 1 + row_base
                    ccarry = ccarry + jnp.sum(ind)
                gathers[b] = pltpu.async_copy(
                    x_hbm.at[g_v.at[cc]], bufs[b], rsems[b])
                if cc >= 1:
                    pb = 1 - b
                    gathers[pb].wait()
                    writes[pb] = pltpu.async_copy(
                        bufs[pb], out_slice(cc - 1), wsems[pb])

            last = (_NCHUNK - 1) % 2
            gathers[last].wait()
            writes[last] = pltpu.async_copy(
                bufs[last], out_slice(_NCHUNK - 1), wsems[last])
            writes[0].wait()
            writes[1].wait()
            return 0, 0

        return lax.cond(contig, contig_path, ragged_path, 0)

    ok, start = lax.cond(allone, fast_path, general_path, 0)

    lane = lax.iota(jnp.int32, _L)
    meta_v[...] = jnp.where(lane == 0, ok, jnp.where(lane == 1, start, 0))
    pltpu.sync_copy(meta_v, meta_hbm.at[wid])


_sc_stage = functools.partial(
    pl.kernel,
    out_type=(
        jax.ShapeDtypeStruct((_B * _T, _D), jnp.float32),
        jax.ShapeDtypeStruct((_NW, _L), jnp.int32),
    ),
    mesh=plsc.VectorSubcoreMesh(
        core_axis_name="c", subcore_axis_name="s",
        num_cores=_NC, num_subcores=_NS),
    compiler_params=pltpu.CompilerParams(needs_layout_passes=False),
    scratch_types=[
        pltpu.VMEM((_T,), jnp.float32),
        pltpu.VMEM((_T,), jnp.int32),
        pltpu.VMEM((_NCHUNK, _CHUNK), jnp.int32),
        pltpu.VMEM((_L,), jnp.int32),
        pltpu.VMEM((_CHUNK, _D), jnp.float32),
        pltpu.VMEM((_CHUNK, _D), jnp.float32),
        pltpu.SemaphoreType.DMA,
        pltpu.SemaphoreType.DMA,
        pltpu.SemaphoreType.DMA,
        pltpu.SemaphoreType.DMA,
    ],
)(_sc_body)


_TCCH = 512
_TCN = (_B * _T) // _TCCH
_TCDEPTH = 4
_TCSUB = _HALF // _TCCH


def _tc_body(meta_ref, x_ref, part_ref, out_ref, *scratch):
    del part_ref
    bufs = scratch[:_TCDEPTH]
    sin = scratch[_TCDEPTH:2 * _TCDEPTH]
    sout = scratch[2 * _TCDEPTH:3 * _TCDEPTH]
    oks, descs_in, descs_out = [], [], []
    for c in range(_TCN):
        b = c % _TCDEPTH
        w, sub = c // _TCSUB, c % _TCSUB
        oks.append(meta_ref[w, 0] == 1)
        start = pl.multiple_of(meta_ref[w, 1] + sub * _TCCH, 8)
        descs_in.append(pltpu.make_async_copy(
            x_ref.at[pl.ds(start, _TCCH)], bufs[b], sin[b]))
        descs_out.append(pltpu.make_async_copy(
            bufs[b], out_ref.at[pl.ds(c * _TCCH, _TCCH)], sout[b]))

    def when_do(pred, fn):
        pl.when(pred)(fn)

    for c in range(_TCN):
        if c >= _TCDEPTH:
            when_do(oks[c - _TCDEPTH],
                    lambda c=c: descs_out[c - _TCDEPTH].wait())
        when_do(oks[c], lambda c=c: descs_in[c].start())
        if c >= 1:
            def retire(c=c):
                descs_in[c - 1].wait()
                descs_out[c - 1].start()
            when_do(oks[c - 1], retire)

    def retire_last():
        descs_in[_TCN - 1].wait()
        descs_out[_TCN - 1].start()

    when_do(oks[_TCN - 1], retire_last)
    for c in range(_TCN - _TCDEPTH, _TCN):
        when_do(oks[c], lambda c=c: descs_out[c].wait())


_tc_stage = pl.pallas_call(
    _tc_body,
    out_shape=jax.ShapeDtypeStruct((_B * _T, _D), jnp.float32),
    in_specs=[
        pl.BlockSpec(memory_space=pltpu.SMEM),
        pl.BlockSpec(memory_space=pl.ANY),
        pl.BlockSpec(memory_space=pl.ANY),
    ],
    out_specs=pl.BlockSpec(memory_space=pl.ANY),
    scratch_shapes=(
        [pltpu.VMEM((_TCCH, _D), jnp.float32)] * _TCDEPTH
        + [pltpu.SemaphoreType.DMA] * (2 * _TCDEPTH)
    ),
    input_output_aliases={2: 0},
)


def kernel(x, dur):
    xr = x.reshape(_B * _T, _D)
    partial, meta = _sc_stage(xr, dur)
    out = _tc_stage(meta, xr, partial)
    return out.reshape(_B, _T, _D)


# speedup vs baseline: 1.1653x; 1.1565x over previous
import functools

import jax
import jax.numpy as jnp
from jax import lax
from jax.experimental import pallas as pl
from jax.experimental.pallas import tpu as pltpu
from jax.experimental.pallas import tpu_sc as plsc

_L = 16
_B, _T, _D = 16, 2048, 512
_NC, _NS = 2, 16
_NW = _NC * _NS
_HALF = _T // 2
_CHUNK = 64
_NCHUNK = _HALF // _CHUNK
_VPC = _CHUNK // _L
_NVEC = _T // _L
_WVEC = _HALF // _L


def _reps(d):
    d = jnp.minimum(jnp.maximum(d, 0.0), 4096.0)
    return (d + 0.5).astype(jnp.int32)


def _sc_body(x_hbm, dur_hbm, out_hbm, meta_hbm,
             dur_v, ind_v, g_v, meta_v, buf0, buf1, rs0, rs1, ws0, ws1):
    cid = lax.axis_index("c")
    sid = lax.axis_index("s")
    wid = sid * _NC + cid
    row = wid // 2
    half = wid % 2

    pltpu.sync_copy(dur_hbm.at[row], dur_v)

    row_base = row * _T
    lo = half * _WVEC
    out_base = row_base + half * _HALF

    def allone_body(k, acc):
        r = _reps(dur_v[pl.ds(k * _L, _L)])
        return jnp.logical_and(acc, jnp.all(r == 1))

    allone = lax.fori_loop(0, _NVEC, allone_body, True)

    def fast_path(_):
        return 1, out_base

    def general_path(_):
        zeros = jnp.zeros((_L,), jnp.int32)

        def zero_body(k, carry):
            ind_v[pl.ds(k * _L, _L)] = zeros
            return carry

        lax.fori_loop(0, _NVEC, zero_body, 0)

        ones = jnp.ones((_L,), jnp.int32)

        def scat_body(k, carry):
            r = _reps(dur_v[pl.ds(k * _L, _L)])
            incl = jnp.cumsum(r)
            excl = incl - r + carry
            plsc.addupdate_scatter(ind_v, (excl,), ones, mask=excl < _T)
            return carry + jnp.sum(r)

        lax.fori_loop(0, _NVEC, scat_body, 0)

        def presum_body(k, carry):
            return carry + jnp.sum(ind_v[pl.ds(k * _L, _L)])

        carry = lax.fori_loop(0, lo, presum_body, 0)

        def contig_body(k, acc):
            ind = ind_v[pl.ds(k * _L, _L)]
            return jnp.logical_and(acc, jnp.all(ind == 1))

        contig = lax.fori_loop(lo, lo + _WVEC, contig_body, True)

        def contig_path(_):
            return 1, row_base + carry

        contig = jnp.logical_and(contig, carry % _HALF == 0)

        def ragged_path(_):
            bufs, rsems, wsems = (buf0, buf1), (rs0, rs1), (ws0, ws1)
            gathers = [None, None]
            writes = [None, None]
            ccarry = carry

            def out_slice(cc):
                return out_hbm.at[pl.ds(out_base + cc * _CHUNK, _CHUNK)]

            for cc in range(_NCHUNK):
                b = cc % 2
                if writes[b] is not None:
                    writes[b].wait()
                for o in range(_VPC):
                    k = lo + cc * _VPC + o
                    ind = ind_v[pl.ds(k * _L, _L)]
                    cs = jnp.cumsum(ind) + ccarry
                    g_v[cc, pl.ds(o * _L, _L)] = cs - 1 + row_base
                    ccarry = ccarry + jnp.sum(ind)
                gathers[b] = pltpu.async_copy(
                    x_hbm.at[g_v.at[cc]], bufs[b], rsems[b])
                if cc >= 1:
                    pb = 1 - b
                    gathers[pb].wait()
                    writes[pb] = pltpu.async_copy(
                        bufs[pb], out_slice(cc - 1), wsems[pb])

            last = (_NCHUNK - 1) % 2
            gathers[last].wait()
            writes[last] = pltpu.async_copy(
                bufs[last], out_slice(_NCHUNK - 1), wsems[last])
            writes[0].wait()
            writes[1].wait()
            return 0, 0

        return lax.cond(contig, contig_path, ragged_path, 0)

    ok, start = lax.cond(allone, fast_path, general_path, 0)

    lane = lax.iota(jnp.int32, _L)
    meta_v[...] = jnp.where(lane == 0, ok, jnp.where(lane == 1, start, 0))
    pltpu.sync_copy(meta_v, meta_hbm.at[wid])


_sc_stage = functools.partial(
    pl.kernel,
    out_type=(
        jax.ShapeDtypeStruct((_B * _T, _D), jnp.float32),
        jax.ShapeDtypeStruct((_NW, _L), jnp.int32),
    ),
    mesh=plsc.VectorSubcoreMesh(
        core_axis_name="c", subcore_axis_name="s",
        num_cores=_NC, num_subcores=_NS),
    compiler_params=pltpu.CompilerParams(needs_layout_passes=False),
    scratch_types=[
        pltpu.VMEM((_T,), jnp.float32),
        pltpu.VMEM((_T,), jnp.int32),
        pltpu.VMEM((_NCHUNK, _CHUNK), jnp.int32),
        pltpu.VMEM((_L,), jnp.int32),
        pltpu.VMEM((_CHUNK, _D), jnp.float32),
        pltpu.VMEM((_CHUNK, _D), jnp.float32),
        pltpu.SemaphoreType.DMA,
        pltpu.SemaphoreType.DMA,
        pltpu.SemaphoreType.DMA,
        pltpu.SemaphoreType.DMA,
    ],
)(_sc_body)


def _x_map(c, meta):
    ok = meta[c, 0] == 1
    return jnp.where(ok, meta[c, 1] // _HALF, 0), 0


def _part_map(c, meta):
    ok = meta[c, 0] == 1
    return jnp.where(ok, 0, c), 0


def _out_map(c, meta):
    return c, 0


def _tc_body(meta_ref, x_ref, part_ref, out_ref):
    c = pl.program_id(0)
    ok = meta_ref[c, 0] == 1

    @pl.when(ok)
    def _():
        out_ref[...] = x_ref[...]

    @pl.when(jnp.logical_not(ok))
    def _():
        out_ref[...] = part_ref[...]


_tc_stage = pl.pallas_call(
    _tc_body,
    grid_spec=pltpu.PrefetchScalarGridSpec(
        num_scalar_prefetch=1,
        grid=(_NW,),
        in_specs=[
            pl.BlockSpec((_HALF, _D), _x_map),
            pl.BlockSpec((_HALF, _D), _part_map),
        ],
        out_specs=pl.BlockSpec((_HALF, _D), _out_map),
    ),
    out_shape=jax.ShapeDtypeStruct((_B * _T, _D), jnp.float32),
    input_output_aliases={2: 0},
)


def kernel(x, dur):
    xr = x.reshape(_B * _T, _D)
    partial, meta = _sc_stage(xr, dur)
    out = _tc_stage(meta, xr, partial)
    return out.reshape(_B, _T, _D)


# speedup vs baseline: 1.1709x; 1.0048x over previous
import functools

import jax
import jax.numpy as jnp
from jax import lax
from jax.experimental import pallas as pl
from jax.experimental.pallas import tpu as pltpu
from jax.experimental.pallas import tpu_sc as plsc

_L = 16
_B, _T, _D = 16, 2048, 512
_NC, _NS = 2, 16
_NW = _NC * _NS
_HALF = _T // 2
_CHUNK = 64
_NCHUNK = _HALF // _CHUNK
_VPC = _CHUNK // _L
_NVEC = _T // _L
_WVEC = _HALF // _L


def _reps(d):
    d = jnp.minimum(jnp.maximum(d, 0.0), 4096.0)
    return (d + 0.5).astype(jnp.int32)


def _sc_body(x_hbm, dur_hbm, out_hbm, meta_hbm,
             dur_v, ind_v, g_v, meta_v, buf0, buf1, rs0, rs1, ws0, ws1):
    cid = lax.axis_index("c")
    sid = lax.axis_index("s")
    wid = sid * _NC + cid
    row = wid // 2
    half = wid % 2

    pltpu.sync_copy(dur_hbm.at[row], dur_v)

    row_base = row * _T
    lo = half * _WVEC
    out_base = row_base + half * _HALF

    def allone_body(k, acc):
        for u in range(8):
            r = _reps(dur_v[pl.ds((k * 8 + u) * _L, _L)])
            acc = acc | (r ^ 1)
        return acc

    dev = lax.fori_loop(0, _NVEC // 8, allone_body, jnp.zeros((_L,), jnp.int32))
    allone = jnp.max(dev) == 0

    def fast_path(_):
        return 1, out_base

    def general_path(_):
        zeros = jnp.zeros((_L,), jnp.int32)

        def zero_body(k, carry):
            ind_v[pl.ds(k * _L, _L)] = zeros
            return carry

        lax.fori_loop(0, _NVEC, zero_body, 0)

        ones = jnp.ones((_L,), jnp.int32)

        def scat_body(k, carry):
            r = _reps(dur_v[pl.ds(k * _L, _L)])
            incl = jnp.cumsum(r)
            excl = incl - r + carry
            plsc.addupdate_scatter(ind_v, (excl,), ones, mask=excl < _T)
            return carry + jnp.sum(r)

        lax.fori_loop(0, _NVEC, scat_body, 0)

        def presum_body(k, carry):
            return carry + jnp.sum(ind_v[pl.ds(k * _L, _L)])

        carry = lax.fori_loop(0, lo, presum_body, 0)

        def contig_body(k, acc):
            ind = ind_v[pl.ds(k * _L, _L)]
            return jnp.logical_and(acc, jnp.all(ind == 1))

        contig = lax.fori_loop(lo, lo + _WVEC, contig_body, True)

        def contig_path(_):
            return 1, row_base + carry

        contig = jnp.logical_and(contig, carry % _HALF == 0)

        def ragged_path(_):
            bufs, rsems, wsems = (buf0, buf1), (rs0, rs1), (ws0, ws1)
            gathers = [None, None]
            writes = [None, None]
            ccarry = carry

            def out_slice(cc):
                return out_hbm.at[pl.ds(out_base + cc * _CHUNK, _CHUNK)]

            for cc in range(_NCHUNK):
                b = cc % 2
                if writes[b] is not None:
                    writes[b].wait()
                for o in range(_VPC):
                    k = lo + cc * _VPC + o
                    ind = ind_v[pl.ds(k * _L, _L)]
                    cs = jnp.cumsum(ind) + ccarry
                    g_v[cc, pl.ds(o * _L, _L)] = cs - 1 + row_base
                    ccarry = ccarry + jnp.sum(ind)
                gathers[b] = pltpu.async_copy(
                    x_hbm.at[g_v.at[cc]], bufs[b], rsems[b])
                if cc >= 1:
                    pb = 1 - b
                    gathers[pb].wait()
                    writes[pb] = pltpu.async_copy(
                        bufs[pb], out_slice(cc - 1), wsems[pb])

            last = (_NCHUNK - 1) % 2
            gathers[last].wait()
            writes[last] = pltpu.async_copy(
                bufs[last], out_slice(_NCHUNK - 1), wsems[last])
            writes[0].wait()
            writes[1].wait()
            return 0, 0

        return lax.cond(contig, contig_path, ragged_path, 0)

    ok, start = lax.cond(allone, fast_path, general_path, 0)

    lane = lax.iota(jnp.int32, _L)
    meta_v[...] = jnp.where(lane == 0, ok, jnp.where(lane == 1, start, 0))
    pltpu.sync_copy(meta_v, meta_hbm.at[wid])


_sc_stage = functools.partial(
    pl.kernel,
    out_type=(
        jax.ShapeDtypeStruct((_B * _T, _D), jnp.float32),
        jax.ShapeDtypeStruct((_NW, _L), jnp.int32),
    ),
    mesh=plsc.VectorSubcoreMesh(
        core_axis_name="c", subcore_axis_name="s",
        num_cores=_NC, num_subcores=_NS),
    compiler_params=pltpu.CompilerParams(needs_layout_passes=False),
    scratch_types=[
        pltpu.VMEM((_T,), jnp.float32),
        pltpu.VMEM((_T,), jnp.int32),
        pltpu.VMEM((_NCHUNK, _CHUNK), jnp.int32),
        pltpu.VMEM((_L,), jnp.int32),
        pltpu.VMEM((_CHUNK, _D), jnp.float32),
        pltpu.VMEM((_CHUNK, _D), jnp.float32),
        pltpu.SemaphoreType.DMA,
        pltpu.SemaphoreType.DMA,
        pltpu.SemaphoreType.DMA,
        pltpu.SemaphoreType.DMA,
    ],
)(_sc_body)


def _x_map(c, meta):
    ok = meta[c, 0] == 1
    return jnp.where(ok, meta[c, 1] // _HALF, 0), 0


def _part_map(c, meta):
    ok = meta[c, 0] == 1
    return jnp.where(ok, 0, c), 0


def _out_map(c, meta):
    return c, 0


def _tc_body(meta_ref, x_ref, part_ref, out_ref):
    c = pl.program_id(0)
    ok = meta_ref[c, 0] == 1

    @pl.when(ok)
    def _():
        out_ref[...] = x_ref[...]

    @pl.when(jnp.logical_not(ok))
    def _():
        out_ref[...] = part_ref[...]


_tc_stage = pl.pallas_call(
    _tc_body,
    grid_spec=pltpu.PrefetchScalarGridSpec(
        num_scalar_prefetch=1,
        grid=(_NW,),
        in_specs=[
            pl.BlockSpec((_HALF, _D), _x_map),
            pl.BlockSpec((_HALF, _D), _part_map),
        ],
        out_specs=pl.BlockSpec((_HALF, _D), _out_map),
    ),
    out_shape=jax.ShapeDtypeStruct((_B * _T, _D), jnp.float32),
    input_output_aliases={2: 0},
)


def kernel(x, dur):
    xr = x.reshape(_B * _T, _D)
    partial, meta = _sc_stage(xr, dur)
    out = _tc_stage(meta, xr, partial)
    return out.reshape(_B, _T, _D)


# speedup vs baseline: 1.1955x; 1.0210x over previous
import functools

import jax
import jax.numpy as jnp
from jax import lax
from jax.experimental import pallas as pl
from jax.experimental.pallas import tpu as pltpu
from jax.experimental.pallas import tpu_sc as plsc

_L = 16
_B, _T, _D = 16, 2048, 512
_NC, _NS = 2, 16
_NW = _NC * _NS
_HALF = _T // 2
_CHUNK = 64
_NCHUNK = _HALF // _CHUNK
_VPC = _CHUNK // _L
_NVEC = _T // _L


_LAST = None


def _bcast_last(v):
    idx = jnp.full((_L, 1), _L - 1, jnp.int32)
    dn = lax.GatherDimensionNumbers(
        offset_dims=(), collapsed_slice_dims=(0,), start_index_map=(0,))
    return lax.gather(v, idx, dn, (1,),
                      mode=lax.GatherScatterMode.PROMISE_IN_BOUNDS)


def _body(x_hbm, dur_hbm, out_hbm, dur_v, ind_v, g_v,
          buf0, buf1, rs0, rs1, ws0, ws1, ds0):
    cid = lax.axis_index("c")
    sid = lax.axis_index("s")
    wid = sid * _NC + cid
    row = wid // 2
    half = wid % 2

    dur_cp = pltpu.async_copy(dur_hbm.at[row], dur_v, ds0)

    zeros = jnp.zeros((_L,), jnp.int32)

    def zero_body(k, carry):
        for u in range(8):
            ind_v[pl.ds((k * 8 + u) * _L, _L)] = zeros
        return carry

    lax.fori_loop(0, _NVEC // 8, zero_body, 0)
    dur_cp.wait()

    ones = jnp.ones((_L,), jnp.int32)

    def scat_body(k, carry):
        d = dur_v[pl.ds(k * _L, _L)]
        d = jnp.minimum(jnp.maximum(d, 0.0), 4096.0)
        r = (d + 0.5).astype(jnp.int32)
        incl = jnp.cumsum(r)
        excl = incl - r + carry
        plsc.addupdate_scatter(ind_v, (excl,), ones, mask=excl < _T)
        return carry + _bcast_last(incl)

    carry0 = lax.fori_loop(0, _NVEC, scat_body, zeros)

    row_base = row * _T
    lo = half * (_HALF // _L)
    out_base = row_base + half * _HALF

    def presum_body(k, acc):
        return acc + ind_v[pl.ds(k * _L, _L)]

    acc = lax.fori_loop(0, lo, presum_body, zeros)
    carry = zeros + jnp.sum(acc)

    bufs, rsems, wsems = (buf0, buf1), (rs0, rs1), (ws0, ws1)
    gathers = [None, None]
    writes = [None, None]

    def out_slice(cc):
        return out_hbm.at[pl.ds(out_base + cc * _CHUNK, _CHUNK)]

    for cc in range(_NCHUNK):
        b = cc % 2
        if writes[b] is not None:
            writes[b].wait()
        for o in range(_VPC):
            k = lo + cc * _VPC + o
            ind = ind_v[pl.ds(k * _L, _L)]
            incl = jnp.cumsum(ind)
            g_v[cc, pl.ds(o * _L, _L)] = incl + carry - 1 + row_base
            carry = carry + _bcast_last(incl)
        gathers[b] = pltpu.async_copy(x_hbm.at[g_v.at[cc]], bufs[b], rsems[b])
        if cc >= 1:
            pb = 1 - b
            gathers[pb].wait()
            writes[pb] = pltpu.async_copy(bufs[pb], out_slice(cc - 1), wsems[pb])

    last = (_NCHUNK - 1) % 2
    gathers[last].wait()
    writes[last] = pltpu.async_copy(bufs[last], out_slice(_NCHUNK - 1), wsems[last])
    writes[0].wait()
    writes[1].wait()


_regulate = functools.partial(
    pl.kernel,
    out_type=jax.ShapeDtypeStruct((_B * _T, _D), jnp.float32),
    mesh=plsc.VectorSubcoreMesh(
        core_axis_name="c", subcore_axis_name="s",
        num_cores=_NC, num_subcores=_NS),
    compiler_params=pltpu.CompilerParams(needs_layout_passes=False),
    scratch_types=[
        pltpu.VMEM((_T,), jnp.float32),
        pltpu.VMEM((_T,), jnp.int32),
        pltpu.VMEM((_NCHUNK, _CHUNK), jnp.int32),
        pltpu.VMEM((_CHUNK, _D), jnp.float32),
        pltpu.VMEM((_CHUNK, _D), jnp.float32),
        pltpu.SemaphoreType.DMA,
        pltpu.SemaphoreType.DMA,
        pltpu.SemaphoreType.DMA,
        pltpu.SemaphoreType.DMA,
        pltpu.SemaphoreType.DMA,
    ],
)(_body)


def kernel(x, dur):
    out = _regulate(x.reshape(_B * _T, _D), dur)
    return out.reshape(_B, _T, _D)


# speedup vs baseline: 1.2001x; 1.0038x over previous
import functools

import jax
import jax.numpy as jnp
from jax import lax
from jax.experimental import pallas as pl
from jax.experimental.pallas import tpu as pltpu
from jax.experimental.pallas import tpu_sc as plsc

_L = 16
_B, _T, _D = 16, 2048, 512
_NC, _NS = 2, 16
_NW = _NC * _NS
_HALF = _T // 2
_CHUNK = 64
_NCHUNK = _HALF // _CHUNK
_VPC = _CHUNK // _L
_NVEC = _T // _L


_LAST = None


def _bcast_last(v):
    idx = jnp.full((_L, 1), _L - 1, jnp.int32)
    dn = lax.GatherDimensionNumbers(
        offset_dims=(), collapsed_slice_dims=(0,), start_index_map=(0,))
    return lax.gather(v, idx, dn, (1,),
                      mode=lax.GatherScatterMode.PROMISE_IN_BOUNDS)


def _body(x_hbm, dur_hbm, out_hbm, dur_v, ind_v, g_v,
          buf0, buf1, rs0, rs1, ws0, ws1, ds0):
    cid = lax.axis_index("c")
    sid = lax.axis_index("s")
    wid = sid * _NC + cid
    row = wid // 2
    half = wid % 2

    dur_cp = pltpu.async_copy(dur_hbm.at[row], dur_v, ds0)

    zeros = jnp.zeros((_L,), jnp.int32)

    def zero_body(k, carry):
        for u in range(8):
            ind_v[pl.ds((k * 8 + u) * _L, _L)] = zeros
        return carry

    lax.fori_loop(0, _NVEC // 8, zero_body, 0)
    dur_cp.wait()

    ones = jnp.ones((_L,), jnp.int32)

    def scat_body(k, carry):
        for u in range(4):
            d = dur_v[pl.ds((k * 4 + u) * _L, _L)]
            d = jnp.minimum(jnp.maximum(d, 0.0), 4096.0)
            r = (d + 0.5).astype(jnp.int32)
            incl = jnp.cumsum(r)
            excl = incl - r + carry
            plsc.addupdate_scatter(ind_v, (excl,), ones, mask=excl < _T)
            carry = carry + _bcast_last(incl)
        return carry

    lax.fori_loop(0, _NVEC // 4, scat_body, zeros)

    row_base = row * _T
    lo = half * (_HALF // _L)
    out_base = row_base + half * _HALF

    def presum_body(k, acc):
        return acc + ind_v[pl.ds(k * _L, _L)]

    acc = lax.fori_loop(0, lo, presum_body, zeros)
    carry = zeros + jnp.sum(acc)

    bufs, rsems, wsems = (buf0, buf1), (rs0, rs1), (ws0, ws1)
    gathers = [None, None]
    writes = [None, None]

    def out_slice(cc):
        return out_hbm.at[pl.ds(out_base + cc * _CHUNK, _CHUNK)]

    for cc in range(_NCHUNK):
        b = cc % 2
        if writes[b] is not None:
            writes[b].wait()
        for o in range(_VPC):
            k = lo + cc * _VPC + o
            ind = ind_v[pl.ds(k * _L, _L)]
            incl = jnp.cumsum(ind)
            g_v[cc, pl.ds(o * _L, _L)] = incl + carry - 1 + row_base
            carry = carry + _bcast_last(incl)
        gathers[b] = pltpu.async_copy(x_hbm.at[g_v.at[cc]], bufs[b], rsems[b])
        if cc >= 1:
            pb = 1 - b
            gathers[pb].wait()
            writes[pb] = pltpu.async_copy(bufs[pb], out_slice(cc - 1), wsems[pb])

    last = (_NCHUNK - 1) % 2
    gathers[last].wait()
    writes[last] = pltpu.async_copy(bufs[last], out_slice(_NCHUNK - 1), wsems[last])
    writes[0].wait()
    writes[1].wait()


_regulate = functools.partial(
    pl.kernel,
    out_type=jax.ShapeDtypeStruct((_B * _T, _D), jnp.float32),
    mesh=plsc.VectorSubcoreMesh(
        core_axis_name="c", subcore_axis_name="s",
        num_cores=_NC, num_subcores=_NS),
    compiler_params=pltpu.CompilerParams(needs_layout_passes=False),
    scratch_types=[
        pltpu.VMEM((_T,), jnp.float32),
        pltpu.VMEM((_T,), jnp.int32),
        pltpu.VMEM((_NCHUNK, _CHUNK), jnp.int32),
        pltpu.VMEM((_CHUNK, _D), jnp.float32),
        pltpu.VMEM((_CHUNK, _D), jnp.float32),
        pltpu.SemaphoreType.DMA,
        pltpu.SemaphoreType.DMA,
        pltpu.SemaphoreType.DMA,
        pltpu.SemaphoreType.DMA,
        pltpu.SemaphoreType.DMA,
    ],
)(_body)


def kernel(x, dur):
    out = _regulate(x.reshape(_B * _T, _D), dur)
    return out.reshape(_B, _T, _D)
